# baseline (speedup 1.0000x reference)
import functools

import jax
import jax.numpy as jnp
from jax import lax
from jax.experimental import pallas as pl
from jax.experimental.pallas import tpu as pltpu
from jax.experimental.pallas import tpu_sc as plsc

B, H, W, K = 4, 224, 224, 8
P, C = 100000, 32
NC, NS, L = 2, 16, 16
NW = NC * NS
ROWS = B * H
ROWS_PER_W = ROWS // NW
GROUPS = W // L
OPITCH = W + 1


def _body(idx_hbm, d2_hbm, feat_hbm, out_hbm, idx_v, d2_v, rows_v, out_v,
          sg, si, so):
    cid = lax.axis_index("c")
    sid = lax.axis_index("s")
    wid = sid * NC + cid
    row0 = wid * ROWS_PER_W
    lanes = lax.iota(jnp.int32, 16)
    c_even = lanes * 2
    c_odd = lanes * 2 + 1

    def start_in(r, b):
        pltpu.async_copy(idx_hbm.at[r], idx_v.at[b], si[b])
        pltpu.async_copy(d2_hbm.at[r], d2_v.at[b], si[b])

    def wait_in(r, b):
        pltpu.make_async_copy(idx_hbm.at[r], idx_v.at[b], si[b]).wait()
        pltpu.make_async_copy(d2_hbm.at[r], d2_v.at[b], si[b]).wait()

    def gather_ops(b):
        return [
            (
                feat_hbm.at[idx_v.at[b, k, pl.ds(j * 112, 112)]],
                rows_v.at[b, pl.ds(k * W + j * 112, 112)],
            )
            for k in range(K)
            for j in range(2)
        ]

    def start_gathers(b):
        for src, dst in gather_ops(b):
            pltpu.async_copy(src, dst, sg[b])

    def wait_gathers(b):
        for src, dst in gather_ops(b):
            pltpu.make_async_copy(src, dst, sg[b]).wait()

    def start_out(r, b):
        pltpu.async_copy(out_v.at[b, :, pl.ds(0, W)], out_hbm.at[r], so[b])

    def wait_out(r, b):
        pltpu.make_async_copy(out_v.at[b, :, pl.ds(0, W)], out_hbm.at[r], so[b]).wait()

    def compute(b):
        @pl.loop(0, GROUPS)
        def _group(g):
            w0 = g * L
            wk = [1.0 - d2_v[b, k, pl.ds(w0, 16)] for k in range(K)]
            den = wk[0]
            for k in range(1, K):
                den = den + wk[k]
            inv16 = 1.0 / jnp.maximum(den, 1e-10)
            for w in range(L):
                acc0 = acc1 = None
                for k in range(K):
                    wb = wk[k][w]
                    packed = rows_v[b, k * W + w0 + w, :]
                    r0, r1 = plsc.unpack(
                        packed,
                        format=plsc.PackFormat.INTERLEAVED,
                        preferred_element_type=jnp.float32,
                    )
                    if acc0 is None:
                        acc0, acc1 = wb * r0, wb * r1
                    else:
                        acc0, acc1 = acc0 + wb * r0, acc1 + wb * r1
                invb = inv16[w]
                wvec = jnp.full((16,), w0 + w, jnp.int32)
                plsc.store_scatter(out_v.at[b], [c_even, wvec], acc0 * invb)
                plsc.store_scatter(out_v.at[b], [c_odd, wvec], acc1 * invb)

    pltpu.sync_copy(idx_hbm.at[row0], idx_v.at[0])
    pltpu.sync_copy(d2_hbm.at[row0], d2_v.at[0])
    start_gathers(0)
    start_in(row0 + 1, 1)

    NT = ROWS_PER_W // 2

    @pl.loop(0, NT)
    def _t(t):
        r0 = row0 + 2 * t

        wait_gathers(0)
        wait_in(r0 + 1, 1)
        start_gathers(1)

        @pl.when(t > 0)
        def _():
            wait_out(r0 - 2, 0)
        compute(0)
        start_out(r0, 0)

        @pl.when(t < NT - 1)
        def _():
            start_in(r0 + 2, 0)

        wait_gathers(1)

        @pl.when(t < NT - 1)
        def _():
            wait_in(r0 + 2, 0)
            start_gathers(0)

        @pl.when(t > 0)
        def _():
            wait_out(r0 - 1, 1)
        compute(1)
        start_out(r0 + 1, 1)

        @pl.when(t < NT - 1)
        def _():
            start_in(r0 + 3, 1)

    wait_out(row0 + ROWS_PER_W - 2, 0)
    wait_out(row0 + ROWS_PER_W - 1, 1)


@functools.partial(
    pl.kernel,
    out_type=jax.ShapeDtypeStruct((ROWS, C, W), jnp.float32),
    mesh=plsc.VectorSubcoreMesh(
        core_axis_name="c", subcore_axis_name="s", num_cores=NC, num_subcores=NS
    ),
    scratch_types=[
        pltpu.VMEM((2, K, W), jnp.int32),
        pltpu.VMEM((2, K, W), jnp.float32),
        pltpu.VMEM((2, K * W, C), jnp.bfloat16),
        pltpu.VMEM((2, C, OPITCH), jnp.float32),
        (pltpu.SemaphoreType.DMA, pltpu.SemaphoreType.DMA),
        (pltpu.SemaphoreType.DMA, pltpu.SemaphoreType.DMA),
        (pltpu.SemaphoreType.DMA, pltpu.SemaphoreType.DMA),
    ],
    compiler_params=pltpu.CompilerParams(
        needs_layout_passes=False, use_tc_tiling_on_sc=False
    ),
)
def _render(idx_hbm, d2_hbm, feat_hbm, out_hbm, idx_v, d2_v, rows_v, out_v,
            sg, si, so):
    _body(idx_hbm, d2_hbm, feat_hbm, out_hbm, idx_v, d2_v, rows_v, out_v,
          sg, si, so)


def kernel(idx, dists2, features, zbuf):
    idx3 = idx.transpose(0, 1, 3, 2).reshape(ROWS, K, W)
    d23 = dists2.transpose(0, 1, 3, 2).reshape(ROWS, K, W)
    featsb = features.astype(jnp.bfloat16)
    out3 = _render(idx3, d23, featsb)
    images = out3.reshape(B, H, C, W).transpose(0, 1, 3, 2)
    return images, zbuf
